# baseline (speedup 1.0000x reference)
import functools

import jax
import jax.numpy as jnp
from jax import lax
from jax.experimental import pallas as pl
from jax.experimental.pallas import tpu as pltpu
from jax.experimental.pallas import tpu_sc as plsc

NC = 50000
NV = 50000
NE = 800000
EMB = 32
NH = 17
BLK = 4096
NBLK = 13
NP = NBLK * BLK
STRIPE = NP // 16
EB = 128
EROWS = 6272
EPAD = EROWS * EB - NE
JMAX = EROWS // 32
KCH = 2
JT = EROWS // 16
NOUTT = JT // KCH


def _vmax_body(var_ref, out_ref):
    i = pl.program_id(0)
    x = var_ref[...]
    rows = lax.broadcasted_iota(jnp.int32, (BLK, 1), 0) + i * BLK
    cf = jnp.maximum(jnp.maximum(jnp.abs(x[:, 0:1]), jnp.abs(x[:, 1:2])), 1.0)
    t = jnp.where(rows < NV, jnp.abs(x[:, 2:3] * cf), 0.0)
    m = jnp.max(t, keepdims=True).reshape(1, 1)

    @pl.when(i == 0)
    def _():
        out_ref[...] = m

    @pl.when(i > 0)
    def _():
        out_ref[...] = jnp.maximum(out_ref[...], m)


def _vmax(var):
    return pl.pallas_call(
        _vmax_body,
        grid=(NBLK,),
        in_specs=[pl.BlockSpec((BLK, 6), lambda i: (i, 0))],
        out_specs=pl.BlockSpec((1, 1), lambda i: (0, 0)),
        out_shape=jax.ShapeDtypeStruct((1, 1), jnp.float32),
    )(var)


def _vembed_body(var_ref, vmax_ref, g_ref, b_ref, w1_ref, b1_ref, w2_ref,
                 b2_ref, out_ref):
    i = pl.program_id(0)
    x = var_ref[...]
    v2m = vmax_ref[...]
    cf = jnp.maximum(jnp.maximum(jnp.abs(x[:, 0:1]), jnp.abs(x[:, 1:2])), 1.0)
    cols = lax.broadcasted_iota(jnp.int32, (BLK, 6), 1)
    vn = jnp.where(cols < 2, x / cf, jnp.where(cols == 2, x * cf / v2m, x))
    m = jnp.mean(vn, axis=1, keepdims=True)
    v = jnp.mean((vn - m) ** 2, axis=1, keepdims=True)
    y = (vn - m) / jnp.sqrt(v + 1e-5) * g_ref[...] + b_ref[...]
    h = jnp.maximum(
        jnp.dot(y, w1_ref[...], preferred_element_type=jnp.float32)
        + b1_ref[...], 0.0)
    e = jnp.maximum(
        jnp.dot(h, w2_ref[...], preferred_element_type=jnp.float32)
        + b2_ref[...], 0.0)
    rows = lax.broadcasted_iota(jnp.int32, (BLK, 1), 0) + i * BLK
    out_ref[...] = jnp.where(rows < NV, e, 0.0)


def _vembed(var, vmax, g, b, w1, b1, w2, b2):
    full = lambda i: (0, 0)
    return pl.pallas_call(
        _vembed_body,
        grid=(NBLK,),
        in_specs=[
            pl.BlockSpec((BLK, 6), lambda i: (i, 0)),
            pl.BlockSpec((1, 1), full),
            pl.BlockSpec((1, 6), full),
            pl.BlockSpec((1, 6), full),
            pl.BlockSpec((6, EMB), full),
            pl.BlockSpec((1, EMB), full),
            pl.BlockSpec((EMB, EMB), full),
            pl.BlockSpec((1, EMB), full),
        ],
        out_specs=pl.BlockSpec((BLK, EMB), lambda i: (i, 0)),
        out_shape=jax.ShapeDtypeStruct((NP, EMB), jnp.float32),
    )(var, vmax, g, b, w1, b1, w2, b2)


_SC_MESH = plsc.VectorSubcoreMesh(core_axis_name="c", subcore_axis_name="s",
                                  num_cores=1)


@functools.partial(
    pl.kernel,
    out_type=(
        jax.ShapeDtypeStruct((NP, EMB), jnp.float32),
        jax.ShapeDtypeStruct((NP,), jnp.float32),
        jax.ShapeDtypeStruct((NP,), jnp.float32),
    ),
    mesh=_SC_MESH,
    compiler_params=pltpu.CompilerParams(use_tc_tiling_on_sc=False),
    scratch_types=[
        pltpu.VMEM_SHARED((NP, EMB), jnp.float32),
        pltpu.VMEM_SHARED((NP,), jnp.float32),
        pltpu.VMEM_SHARED((NP,), jnp.float32),
        pltpu.VMEM((2, KCH, EB), jnp.int32),
        pltpu.VMEM((2, KCH, EB), jnp.int32),
        pltpu.VMEM((2, KCH, EB, EMB), jnp.float32),
        pltpu.VMEM((EB,), jnp.float32),
        pltpu.SemaphoreType.DMA,
        pltpu.SemaphoreType.DMA,
        pltpu.SemaphoreType.DMA,
        pltpu.SemaphoreType.DMA,
    ],
)
def _edge_agg(src_hbm, dst_hbm, ve_hbm, z2d_hbm, z1d_hbm,
              s_out, degc_out, degv_out,
              s_sh, degc_sh, degv_sh, srcb, dstb, rows, ones,
              gsem_a, gsem_b, ssem_a, ssem_b):
    sid = lax.axis_index("s")
    gsems = (gsem_a, gsem_b)
    ssems = (ssem_a, ssem_b)
    for t in range(EB // 16):
        ones[pl.ds(t * 16, 16)] = jnp.ones((16,), jnp.float32)
    stripe = sid * STRIPE
    pltpu.sync_copy(z2d_hbm, s_sh.at[pl.ds(stripe, STRIPE)])
    pltpu.sync_copy(z1d_hbm, degc_sh.at[pl.ds(stripe, STRIPE)])
    pltpu.sync_copy(z1d_hbm, degv_sh.at[pl.ds(stripe, STRIPE)])
    plsc.subcore_barrier()

    def _pipeline(src_hbm, dst_hbm, ve_hbm):
        def _fire_gathers(o, p):
            pltpu.sync_copy(src_hbm.at[sid, pl.ds(o * KCH, KCH)], srcb.at[p])
            pltpu.sync_copy(dst_hbm.at[sid, pl.ds(o * KCH, KCH)], dstb.at[p])
            for q in range(KCH):
                pltpu.async_copy(ve_hbm.at[srcb.at[p, q]], rows.at[p, q],
                                 gsems[p])

        def _fire_scatters(p):
            for q in range(KCH):
                pltpu.async_copy(rows.at[p, q], s_sh.at[dstb.at[p, q]],
                                 ssems[p], add=True)
                pltpu.async_copy(ones, degc_sh.at[dstb.at[p, q]], ssems[p],
                                 add=True)
                pltpu.async_copy(ones, degv_sh.at[srcb.at[p, q]], ssems[p],
                                 add=True)

        def _drain_gathers(p):
            for q in range(KCH):
                pltpu.make_async_copy(ve_hbm.at[srcb.at[p, q]],
                                      rows.at[p, q], gsems[p]).wait()

        def _drain_scatters(p):
            for q in range(KCH):
                pltpu.make_async_copy(rows.at[p, q], s_sh.at[dstb.at[p, q]],
                                      ssems[p]).wait()
                pltpu.make_async_copy(ones, degc_sh.at[dstb.at[p, q]],
                                      ssems[p]).wait()
                pltpu.make_async_copy(ones, degv_sh.at[srcb.at[p, q]],
                                      ssems[p]).wait()

        def _step(o, p):
            @pl.when(o >= 2)
            def _():
                _drain_scatters(p)

            _fire_gathers(o, p)

            @pl.when(o >= 1)
            def _():
                _drain_gathers(1 - p)
                _fire_scatters(1 - p)

        def outer(oo, carry):
            _step(oo * 2, 0)
            _step(oo * 2 + 1, 1)
            return carry

        lax.fori_loop(0, NOUTT // 2, outer, 0)
        _drain_gathers(1)
        _fire_scatters(1)
        _drain_scatters(0)
        _drain_scatters(1)

    _pipeline(src_hbm, dst_hbm, ve_hbm)

    plsc.subcore_barrier()
    pltpu.sync_copy(s_sh.at[pl.ds(stripe, STRIPE)],
                    s_out.at[pl.ds(stripe, STRIPE)])
    pltpu.sync_copy(degc_sh.at[pl.ds(stripe, STRIPE)],
                    degc_out.at[pl.ds(stripe, STRIPE)])
    pltpu.sync_copy(degv_sh.at[pl.ds(stripe, STRIPE)],
                    degv_out.at[pl.ds(stripe, STRIPE)])


def _pool_body(s_ref, dc_ref, dv_ref, ve_ref,
               wm_ref, ws_ref, kv_ref, sc_ref, mv_ref, bs_ref, out_ref):
    i = pl.program_id(0)
    rows = lax.broadcasted_iota(jnp.int32, (BLK, 1), 0) + i * BLK
    valid = rows < NC
    cons = jnp.maximum(
        jnp.dot(s_ref[0], wm_ref[...], preferred_element_type=jnp.float32)
        + dc_ref[0] * kv_ref[...] + sc_ref[...], 0.0)
    cons = jnp.where(valid, cons, 0.0)
    varc = jnp.maximum(
        dv_ref[0] * mv_ref[...]
        + jnp.dot(ve_ref[...], ws_ref[...], preferred_element_type=jnp.float32)
        + bs_ref[...], 0.0)
    varc = jnp.where(valid, varc, 0.0)
    part = jnp.concatenate(
        [jnp.sum(varc, axis=0, keepdims=True),
         jnp.sum(cons, axis=0, keepdims=True)], axis=1)

    @pl.when(i == 0)
    def _():
        out_ref[...] = part

    @pl.when(i > 0)
    def _():
        out_ref[...] = out_ref[...] + part


def _pool(s3, dc3, dv3, ve, wm, ws, kv, sc, mv, bs):
    full = lambda i: (0, 0)
    col = lambda i: (i, 0)
    gcol = lambda i: (0, i, 0)
    return pl.pallas_call(
        _pool_body,
        grid=(NBLK,),
        in_specs=[
            pl.BlockSpec((1, BLK, EMB), gcol),
            pl.BlockSpec((1, BLK, 1), gcol),
            pl.BlockSpec((1, BLK, 1), gcol),
            pl.BlockSpec((BLK, EMB), col),
            pl.BlockSpec((EMB, NH), full),
            pl.BlockSpec((EMB, NH), full),
            pl.BlockSpec((1, NH), full),
            pl.BlockSpec((1, NH), full),
            pl.BlockSpec((1, NH), full),
            pl.BlockSpec((1, NH), full),
        ],
        out_specs=pl.BlockSpec((1, 2 * NH), full),
        out_shape=jax.ShapeDtypeStruct((1, 2 * NH), jnp.float32),
    )(s3, dc3, dv3, ve, wm, ws, kv, sc, mv, bs)


def _head_body(s0_ref, s1_ref, g_ref, b_ref, w1_ref, b1_ref, w2_ref, out_ref):
    x = (s1_ref[...] - s0_ref[...]) * (1.0 / NC)
    m = jnp.mean(x, axis=1, keepdims=True)
    v = jnp.mean((x - m) ** 2, axis=1, keepdims=True)
    y = (x - m) / jnp.sqrt(v + 1e-5) * g_ref[...] + b_ref[...]
    h = jnp.maximum(
        jnp.dot(y, w1_ref[...], preferred_element_type=jnp.float32)
        + b1_ref[...], 0.0)
    o = jnp.dot(h, w2_ref[...], preferred_element_type=jnp.float32)
    out_ref[...] = jax.nn.sigmoid(o)


def _head(s0, s1, g, b, w1, b1, w2):
    D = 2 * NH
    full = lambda: (0, 0)
    return pl.pallas_call(
        _head_body,
        in_specs=[
            pl.BlockSpec((1, D), full),
            pl.BlockSpec((1, D), full),
            pl.BlockSpec((1, D), full),
            pl.BlockSpec((1, D), full),
            pl.BlockSpec((D, 128), full),
            pl.BlockSpec((1, 128), full),
            pl.BlockSpec((128, 1), full),
        ],
        out_specs=pl.BlockSpec((1, 1), full),
        out_shape=jax.ShapeDtypeStruct((1, 1), jnp.float32),
    )(s0, s1, g, b, w1, b1, w2)


def kernel(constraint_features_s, edge_index_s, edge_attr_s,
           variable_features_s, constraint_features_t, edge_index_t,
           edge_attr_t, variable_features_t, params):
    p = params
    wm = jnp.concatenate([p['conv%d_msg_w' % i] for i in range(3)], axis=1)
    bm = jnp.concatenate([p['conv%d_msg_b' % i] for i in range(3)])
    we = jnp.concatenate([p['conv%d_edge_w' % i] for i in range(3)], axis=1)
    be = jnp.concatenate([p['conv%d_edge_b' % i] for i in range(3)])
    ws = jnp.concatenate([p['conv%d_self_w' % i] for i in range(3)], axis=1)
    bs = jnp.concatenate([p['conv%d_self_b' % i] for i in range(3)])
    ce = jax.nn.relu(p['cons_ln_b'] @ p['cons_w1'] + p['cons_b1'])
    cons_row = jax.nn.relu(ce @ p['cons_w2'] + p['cons_b2'])
    kv = bm + p['edge_ln_b'] @ we + be
    sc = cons_row @ ws + bs
    mv = cons_row @ wm + kv
    kv, sc, mv, bs2 = (a.reshape(1, NH) for a in (kv, sc, mv, bs))
    gln = p['var_ln_g'].reshape(1, 6)
    bln = p['var_ln_b'].reshape(1, 6)
    b1 = p['var_b1'].reshape(1, EMB)
    b2 = p['var_b2'].reshape(1, EMB)

    fill = 50000 + jnp.arange(EPAD, dtype=jnp.int32) % (NP - 50000)
    z2d = jnp.zeros((STRIPE, EMB), jnp.float32)
    z1d = jnp.zeros((STRIPE,), jnp.float32)

    pooled = []
    for ei, var in ((edge_index_s, variable_features_s),
                    (edge_index_t, variable_features_t)):
        vmax = _vmax(var)
        ve = _vembed(var, vmax, gln, bln, p['var_w1'], b1, p['var_w2'], b2)
        src3d = jnp.concatenate([ei[0], fill]).reshape(16, JT, EB)
        dst3d = jnp.concatenate([ei[1], fill]).reshape(16, JT, EB)
        s, degc, degv = _edge_agg(src3d, dst3d, ve, z2d, z1d)
        pooled.append(_pool(s.reshape(1, NP, EMB), degc.reshape(1, NP, 1),
                            degv.reshape(1, NP, 1), ve,
                            wm, ws, kv, sc, mv, bs2))
    out = _head(pooled[0], pooled[1],
                p['fin_ln_g'].reshape(1, 2 * NH),
                p['fin_ln_b'].reshape(1, 2 * NH),
                p['fin_w1'], p['fin_b1'].reshape(1, 128), p['fin_w2'])
    return out.reshape(1)


# speedup vs baseline: 1.5263x; 1.5263x over previous
import functools

import jax
import jax.numpy as jnp
from jax import lax
from jax.experimental import pallas as pl
from jax.experimental.pallas import tpu as pltpu
from jax.experimental.pallas import tpu_sc as plsc

NC = 50000
NV = 50000
NE = 800000
EMB = 32
NH = 17
BLK = 4096
NBLK = 13
NP = NBLK * BLK
STRIPE = NP // 16
EB = 128
EROWS = 6272
EPAD = EROWS * EB - NE
JMAX = EROWS // 32
KCH = 14
JT = EROWS // 16
NOUTT = JT // KCH


def _vmax_body(var_ref, out_ref):
    i = pl.program_id(0)
    x = var_ref[...]
    rows = lax.broadcasted_iota(jnp.int32, (BLK, 1), 0) + i * BLK
    cf = jnp.maximum(jnp.maximum(jnp.abs(x[:, 0:1]), jnp.abs(x[:, 1:2])), 1.0)
    t = jnp.where(rows < NV, jnp.abs(x[:, 2:3] * cf), 0.0)
    m = jnp.max(t, keepdims=True).reshape(1, 1)

    @pl.when(i == 0)
    def _():
        out_ref[...] = m

    @pl.when(i > 0)
    def _():
        out_ref[...] = jnp.maximum(out_ref[...], m)


def _vmax(var):
    return pl.pallas_call(
        _vmax_body,
        grid=(NBLK,),
        in_specs=[pl.BlockSpec((BLK, 6), lambda i: (i, 0))],
        out_specs=pl.BlockSpec((1, 1), lambda i: (0, 0)),
        out_shape=jax.ShapeDtypeStruct((1, 1), jnp.float32),
    )(var)


def _vembed_body(var_ref, vmax_ref, g_ref, b_ref, w1_ref, b1_ref, w2_ref,
                 b2_ref, out_ref):
    i = pl.program_id(0)
    x = var_ref[...]
    v2m = vmax_ref[...]
    cf = jnp.maximum(jnp.maximum(jnp.abs(x[:, 0:1]), jnp.abs(x[:, 1:2])), 1.0)
    cols = lax.broadcasted_iota(jnp.int32, (BLK, 6), 1)
    vn = jnp.where(cols < 2, x / cf, jnp.where(cols == 2, x * cf / v2m, x))
    m = jnp.mean(vn, axis=1, keepdims=True)
    v = jnp.mean((vn - m) ** 2, axis=1, keepdims=True)
    y = (vn - m) / jnp.sqrt(v + 1e-5) * g_ref[...] + b_ref[...]
    h = jnp.maximum(
        jnp.dot(y, w1_ref[...], preferred_element_type=jnp.float32)
        + b1_ref[...], 0.0)
    e = jnp.maximum(
        jnp.dot(h, w2_ref[...], preferred_element_type=jnp.float32)
        + b2_ref[...], 0.0)
    rows = lax.broadcasted_iota(jnp.int32, (BLK, 1), 0) + i * BLK
    out_ref[...] = jnp.where(rows < NV, e, 0.0).astype(jnp.bfloat16)


def _vembed(var, vmax, g, b, w1, b1, w2, b2):
    full = lambda i: (0, 0)
    return pl.pallas_call(
        _vembed_body,
        grid=(NBLK,),
        in_specs=[
            pl.BlockSpec((BLK, 6), lambda i: (i, 0)),
            pl.BlockSpec((1, 1), full),
            pl.BlockSpec((1, 6), full),
            pl.BlockSpec((1, 6), full),
            pl.BlockSpec((6, EMB), full),
            pl.BlockSpec((1, EMB), full),
            pl.BlockSpec((EMB, EMB), full),
            pl.BlockSpec((1, EMB), full),
        ],
        out_specs=pl.BlockSpec((BLK, EMB), lambda i: (i, 0)),
        out_shape=jax.ShapeDtypeStruct((NP, EMB), jnp.bfloat16),
    )(var, vmax, g, b, w1, b1, w2, b2)


_SC_MESH = plsc.VectorSubcoreMesh(core_axis_name="c", subcore_axis_name="s")


@functools.partial(
    pl.kernel,
    out_type=(
        jax.ShapeDtypeStruct((2, NP, EMB), jnp.bfloat16),
        jax.ShapeDtypeStruct((2, NP), jnp.float32),
        jax.ShapeDtypeStruct((2, NP), jnp.float32),
    ),
    mesh=_SC_MESH,
    compiler_params=pltpu.CompilerParams(use_tc_tiling_on_sc=False),
    scratch_types=[
        pltpu.VMEM_SHARED((NP, EMB), jnp.bfloat16),
        pltpu.VMEM_SHARED((NP,), jnp.float32),
        pltpu.VMEM_SHARED((NP,), jnp.float32),
        pltpu.VMEM((2, KCH, EB), jnp.int32),
        pltpu.VMEM((2, KCH, EB), jnp.int32),
        pltpu.VMEM((2, KCH, EB, EMB), jnp.bfloat16),
        pltpu.VMEM((EB,), jnp.float32),
        pltpu.SemaphoreType.DMA,
        pltpu.SemaphoreType.DMA,
        pltpu.SemaphoreType.DMA,
        pltpu.SemaphoreType.DMA,
    ],
)
def _edge_agg(src_s, dst_s, ve_s, src_t, dst_t, ve_t, z2d_hbm, z1d_hbm,
              s_out, degc_out, degv_out,
              s_sh, degc_sh, degv_sh, srcb, dstb, rows, ones,
              gsem_a, gsem_b, ssem_a, ssem_b):
    cid = lax.axis_index("c")
    sid = lax.axis_index("s")
    gsems = (gsem_a, gsem_b)
    ssems = (ssem_a, ssem_b)
    for t in range(EB // 16):
        ones[pl.ds(t * 16, 16)] = jnp.ones((16,), jnp.float32)
    stripe = sid * STRIPE
    pltpu.sync_copy(z2d_hbm, s_sh.at[pl.ds(stripe, STRIPE)])
    pltpu.sync_copy(z1d_hbm, degc_sh.at[pl.ds(stripe, STRIPE)])
    pltpu.sync_copy(z1d_hbm, degv_sh.at[pl.ds(stripe, STRIPE)])
    plsc.subcore_barrier()

    def _pipeline(src_hbm, dst_hbm, ve_hbm):
        def _fire_gathers(o, p):
            pltpu.sync_copy(src_hbm.at[sid, pl.ds(o * KCH, KCH)], srcb.at[p])
            pltpu.sync_copy(dst_hbm.at[sid, pl.ds(o * KCH, KCH)], dstb.at[p])
            for q in range(KCH):
                pltpu.async_copy(ve_hbm.at[srcb.at[p, q]], rows.at[p, q],
                                 gsems[p])

        def _fire_scatters(p):
            for q in range(KCH):
                pltpu.async_copy(rows.at[p, q], s_sh.at[dstb.at[p, q]],
                                 ssems[p], add=True)
                pltpu.async_copy(ones, degc_sh.at[dstb.at[p, q]], ssems[p],
                                 add=True)
                pltpu.async_copy(ones, degv_sh.at[srcb.at[p, q]], ssems[p],
                                 add=True)

        def _drain_gathers(p):
            for q in range(KCH):
                pltpu.make_async_copy(ve_hbm.at[srcb.at[p, q]],
                                      rows.at[p, q], gsems[p]).wait()

        def _drain_scatters(p):
            for q in range(KCH):
                pltpu.make_async_copy(rows.at[p, q], s_sh.at[dstb.at[p, q]],
                                      ssems[p]).wait()
                pltpu.make_async_copy(ones, degc_sh.at[dstb.at[p, q]],
                                      ssems[p]).wait()
                pltpu.make_async_copy(ones, degv_sh.at[srcb.at[p, q]],
                                      ssems[p]).wait()

        def _step(o, p):
            @pl.when(o >= 2)
            def _():
                _drain_scatters(p)

            _fire_gathers(o, p)

            @pl.when(o >= 1)
            def _():
                _drain_gathers(1 - p)
                _fire_scatters(1 - p)

        def outer(oo, carry):
            _step(oo * 2, 0)
            _step(oo * 2 + 1, 1)
            return carry

        lax.fori_loop(0, NOUTT // 2, outer, 0)
        _drain_gathers(1)
        _fire_scatters(1)
        _drain_scatters(0)
        _drain_scatters(1)

    @pl.when(cid == 0)
    def _():
        _pipeline(src_s, dst_s, ve_s)

    @pl.when(cid == 1)
    def _():
        _pipeline(src_t, dst_t, ve_t)

    plsc.subcore_barrier()
    pltpu.sync_copy(s_sh.at[pl.ds(stripe, STRIPE)],
                    s_out.at[cid, pl.ds(stripe, STRIPE)])
    pltpu.sync_copy(degc_sh.at[pl.ds(stripe, STRIPE)],
                    degc_out.at[cid, pl.ds(stripe, STRIPE)])
    pltpu.sync_copy(degv_sh.at[pl.ds(stripe, STRIPE)],
                    degv_out.at[cid, pl.ds(stripe, STRIPE)])


def _pool_body(s_ref, dc_ref, dv_ref, ve_ref,
               wm_ref, ws_ref, kv_ref, sc_ref, mv_ref, bs_ref, out_ref):
    i = pl.program_id(0)
    rows = lax.broadcasted_iota(jnp.int32, (BLK, 1), 0) + i * BLK
    valid = rows < NC
    cons = jnp.maximum(
        jnp.dot(s_ref[0].astype(jnp.float32), wm_ref[...],
                preferred_element_type=jnp.float32)
        + dc_ref[0] * kv_ref[...] + sc_ref[...], 0.0)
    cons = jnp.where(valid, cons, 0.0)
    varc = jnp.maximum(
        dv_ref[0] * mv_ref[...]
        + jnp.dot(ve_ref[...].astype(jnp.float32), ws_ref[...],
                  preferred_element_type=jnp.float32)
        + bs_ref[...], 0.0)
    varc = jnp.where(valid, varc, 0.0)
    part = jnp.concatenate(
        [jnp.sum(varc, axis=0, keepdims=True),
         jnp.sum(cons, axis=0, keepdims=True)], axis=1)

    @pl.when(i == 0)
    def _():
        out_ref[...] = part

    @pl.when(i > 0)
    def _():
        out_ref[...] = out_ref[...] + part


def _pool(g, s3, dc3, dv3, ve, wm, ws, kv, sc, mv, bs):
    full = lambda i: (0, 0)
    col = lambda i: (i, 0)
    gcol = lambda i: (g, i, 0)
    return pl.pallas_call(
        _pool_body,
        grid=(NBLK,),
        in_specs=[
            pl.BlockSpec((1, BLK, EMB), gcol),
            pl.BlockSpec((1, BLK, 1), gcol),
            pl.BlockSpec((1, BLK, 1), gcol),
            pl.BlockSpec((BLK, EMB), col),
            pl.BlockSpec((EMB, NH), full),
            pl.BlockSpec((EMB, NH), full),
            pl.BlockSpec((1, NH), full),
            pl.BlockSpec((1, NH), full),
            pl.BlockSpec((1, NH), full),
            pl.BlockSpec((1, NH), full),
        ],
        out_specs=pl.BlockSpec((1, 2 * NH), full),
        out_shape=jax.ShapeDtypeStruct((1, 2 * NH), jnp.float32),
    )(s3, dc3, dv3, ve, wm, ws, kv, sc, mv, bs)


def _head_body(s0_ref, s1_ref, g_ref, b_ref, w1_ref, b1_ref, w2_ref, out_ref):
    x = (s1_ref[...] - s0_ref[...]) * (1.0 / NC)
    m = jnp.mean(x, axis=1, keepdims=True)
    v = jnp.mean((x - m) ** 2, axis=1, keepdims=True)
    y = (x - m) / jnp.sqrt(v + 1e-5) * g_ref[...] + b_ref[...]
    h = jnp.maximum(
        jnp.dot(y, w1_ref[...], preferred_element_type=jnp.float32)
        + b1_ref[...], 0.0)
    o = jnp.dot(h, w2_ref[...], preferred_element_type=jnp.float32)
    out_ref[...] = jax.nn.sigmoid(o)


def _head(s0, s1, g, b, w1, b1, w2):
    D = 2 * NH
    full = lambda: (0, 0)
    return pl.pallas_call(
        _head_body,
        in_specs=[
            pl.BlockSpec((1, D), full),
            pl.BlockSpec((1, D), full),
            pl.BlockSpec((1, D), full),
            pl.BlockSpec((1, D), full),
            pl.BlockSpec((D, 128), full),
            pl.BlockSpec((1, 128), full),
            pl.BlockSpec((128, 1), full),
        ],
        out_specs=pl.BlockSpec((1, 1), full),
        out_shape=jax.ShapeDtypeStruct((1, 1), jnp.float32),
    )(s0, s1, g, b, w1, b1, w2)


def kernel(constraint_features_s, edge_index_s, edge_attr_s,
           variable_features_s, constraint_features_t, edge_index_t,
           edge_attr_t, variable_features_t, params):
    p = params
    wm = jnp.concatenate([p['conv%d_msg_w' % i] for i in range(3)], axis=1)
    bm = jnp.concatenate([p['conv%d_msg_b' % i] for i in range(3)])
    we = jnp.concatenate([p['conv%d_edge_w' % i] for i in range(3)], axis=1)
    be = jnp.concatenate([p['conv%d_edge_b' % i] for i in range(3)])
    ws = jnp.concatenate([p['conv%d_self_w' % i] for i in range(3)], axis=1)
    bs = jnp.concatenate([p['conv%d_self_b' % i] for i in range(3)])
    ce = jax.nn.relu(p['cons_ln_b'] @ p['cons_w1'] + p['cons_b1'])
    cons_row = jax.nn.relu(ce @ p['cons_w2'] + p['cons_b2'])
    kv = bm + p['edge_ln_b'] @ we + be
    sc = cons_row @ ws + bs
    mv = cons_row @ wm + kv
    kv, sc, mv, bs2 = (a.reshape(1, NH) for a in (kv, sc, mv, bs))
    gln = p['var_ln_g'].reshape(1, 6)
    bln = p['var_ln_b'].reshape(1, 6)
    b1 = p['var_b1'].reshape(1, EMB)
    b2 = p['var_b2'].reshape(1, EMB)

    fill = 50000 + jnp.arange(EPAD, dtype=jnp.int32) % (NP - 50000)
    z2d = jnp.zeros((STRIPE, EMB), jnp.bfloat16)
    z1d = jnp.zeros((STRIPE,), jnp.float32)

    ves, srcs, dsts = [], [], []
    for ei, var in ((edge_index_s, variable_features_s),
                    (edge_index_t, variable_features_t)):
        vmax = _vmax(var)
        ves.append(_vembed(var, vmax, gln, bln, p['var_w1'], b1,
                           p['var_w2'], b2))
        srcs.append(jnp.concatenate([ei[0], fill]).reshape(16, JT, EB))
        dsts.append(jnp.concatenate([ei[1], fill]).reshape(16, JT, EB))
    s3, degc, degv = _edge_agg(srcs[0], dsts[0], ves[0],
                               srcs[1], dsts[1], ves[1], z2d, z1d)
    dc3 = degc.reshape(2, NP, 1)
    dv3 = degv.reshape(2, NP, 1)
    pooled = [_pool(g, s3, dc3, dv3, ves[g], wm, ws, kv, sc, mv, bs2)
              for g in (0, 1)]
    out = _head(pooled[0], pooled[1],
                p['fin_ln_g'].reshape(1, 2 * NH),
                p['fin_ln_b'].reshape(1, 2 * NH),
                p['fin_w1'], p['fin_b1'].reshape(1, 128), p['fin_w2'])
    return out.reshape(1)
